# baseline (speedup 1.0000x reference)
import functools

import jax
import jax.numpy as jnp
from jax import lax
from jax.experimental import pallas as pl
from jax.experimental.pallas import tpu as pltpu
from jax.experimental.pallas import tpu_sc as plsc

N_NODES = 1000000
EMBED_DIM = 64
FEAT_DIM = 256
LABELS_SIZE = 64
BATCH = 4096

_LANES = 16
_TILE_W = 128
_CHUNK = 1
_RING = 14


@functools.cache
def _make_sc_gather(V, D, B):
    info = plsc.get_sparse_core_info()
    NC, NS = info.num_cores, info.num_subcores
    NW = NC * NS
    assert B % (8 * NW) == 0
    b_per_w = B // NW
    n_chunks = b_per_w // _CHUNK
    mesh = plsc.VectorSubcoreMesh(core_axis_name="c", subcore_axis_name="s")

    @functools.partial(
        pl.kernel,
        mesh=mesh,
        out_type=jax.ShapeDtypeStruct((D, B), jnp.float32),
        compiler_params=pltpu.CompilerParams(needs_layout_passes=False),
        scratch_types=[
            pltpu.VMEM((b_per_w,), jnp.int32),
            pltpu.VMEM((_RING, _CHUNK, D, _TILE_W), jnp.float32),
            pltpu.VMEM((D, b_per_w), jnp.float32),
        ] + [pltpu.SemaphoreType.DMA] * _RING,
    )
    def gather(tableT_hbm, idx_hbm, outT_hbm, idx_v, bufs, colsT_v, *sems):
        wid = lax.axis_index("s") * NC + lax.axis_index("c")
        base = pl.multiple_of(wid * b_per_w, b_per_w)
        pltpu.sync_copy(idx_hbm.at[pl.ds(base, b_per_w)], idx_v)

        tc_vecs = []
        c_vecs = []
        for q in range(b_per_w // _LANES):
            vec = idx_v[pl.ds(q * _LANES, _LANES)]
            tc_vecs.append(lax.shift_right_logical(vec, 7))
            c_vecs.append(lax.bitwise_and(vec, 127))

        iota = lax.broadcasted_iota(jnp.int32, (_LANES,), 0)

        def issue(g):
            slot = g % _RING
            cps = []
            for b in range(_CHUNK):
                j = g * _CHUNK + b
                q, k = j // _LANES, j % _LANES
                off = pl.multiple_of(tc_vecs[q][k] * _TILE_W, _TILE_W)
                cps.append(pltpu.async_copy(
                    tableT_hbm.at[:, pl.ds(off, _TILE_W)],
                    bufs.at[slot, b], sems[slot]))
            return cps

        pending = {g: issue(g) for g in range(min(_RING - 1, n_chunks))}
        for g in range(n_chunks):
            ahead = g + _RING - 1
            if ahead < n_chunks:
                pending[ahead] = issue(ahead)
            for c in pending.pop(g):
                c.wait()
            slot = g % _RING
            for b in range(_CHUNK):
                j = g * _CHUNK + b
                q, k = j // _LANES, j % _LANES
                cols = jnp.full((_LANES,), c_vecs[q][k], jnp.int32)
                outcols = jnp.full((_LANES,), j, jnp.int32)
                for s in range(D // _LANES):
                    rows = iota + (s * _LANES)
                    vals = plsc.load_gather(bufs.at[slot, b], [rows, cols])
                    plsc.store_scatter(colsT_v, [rows, outcols], vals)

        pltpu.sync_copy(colsT_v, outT_hbm.at[:, pl.ds(base, b_per_w)])

    return gather


def _mlp_a_body(bx_ref, wkT_ref, bk_ref, wpT_ref, bp_ref, pfT_ref):
    wpfT = wpT_ref[:, :LABELS_SIZE]
    h_f = jnp.maximum(
        lax.dot_general(bx_ref[...], wkT_ref[...],
                        dimension_numbers=(((1,), (1,)), ((), ())),
                        preferred_element_type=jnp.float32) + bk_ref[...],
        0.0)
    bp_col = jnp.reshape(bp_ref[...], (LABELS_SIZE, 1))
    pfT_ref[...] = lax.dot_general(
        wpfT, h_f,
        dimension_numbers=(((1,), (1,)), ((), ())),
        preferred_element_type=jnp.float32) + bp_col


def _mlp_b_body(pfT_ref, embsT_ref, wl_ref, bl_ref, wpT_ref, outT_ref):
    wpeT = wpT_ref[:, LABELS_SIZE:]
    h_e = jnp.maximum(
        lax.dot_general(embsT_ref[...], wl_ref[...],
                        dimension_numbers=(((0,), (0,)), ((), ())),
                        preferred_element_type=jnp.float32) + bl_ref[...],
        0.0)
    logitsT = pfT_ref[...] + lax.dot_general(
        wpeT, h_e,
        dimension_numbers=(((1,), (1,)), ((), ())),
        preferred_element_type=jnp.float32)
    m = jnp.max(logitsT, axis=0, keepdims=True)
    e = jnp.exp(logitsT - m)
    outT_ref[...] = e / jnp.sum(e, axis=0, keepdims=True)


def _mlp_a(b_x, WkT, bk, WpT, bp):
    B, F = b_x.shape
    BLK = 2048
    return pl.pallas_call(
        _mlp_a_body,
        grid=(B // BLK,),
        in_specs=[
            pl.BlockSpec((BLK, F), lambda i: (i, 0)),
            pl.BlockSpec((LABELS_SIZE, F), lambda i: (0, 0)),
            pl.BlockSpec((1, LABELS_SIZE), lambda i: (0, 0)),
            pl.BlockSpec((LABELS_SIZE, 2 * LABELS_SIZE), lambda i: (0, 0)),
            pl.BlockSpec((1, LABELS_SIZE), lambda i: (0, 0)),
        ],
        out_specs=pl.BlockSpec((LABELS_SIZE, BLK), lambda i: (0, i)),
        out_shape=jax.ShapeDtypeStruct((LABELS_SIZE, B), jnp.float32),
    )(b_x, WkT, bk.reshape(1, -1), WpT, bp.reshape(1, -1))


def _mlp_b(pfT, embsT, Wl, bl, WpT):
    _, B = pfT.shape
    BLK = 2048
    return pl.pallas_call(
        _mlp_b_body,
        grid=(B // BLK,),
        in_specs=[
            pl.BlockSpec((LABELS_SIZE, BLK), lambda i: (0, i)),
            pl.BlockSpec((EMBED_DIM, BLK), lambda i: (0, i)),
            pl.BlockSpec((EMBED_DIM, LABELS_SIZE), lambda i: (0, 0)),
            pl.BlockSpec((1, LABELS_SIZE), lambda i: (0, 0)),
            pl.BlockSpec((LABELS_SIZE, 2 * LABELS_SIZE), lambda i: (0, 0)),
        ],
        out_specs=pl.BlockSpec((LABELS_SIZE, BLK), lambda i: (0, i)),
        out_shape=jax.ShapeDtypeStruct((LABELS_SIZE, B), jnp.float32),
    )(pfT, embsT, Wl, bl.reshape(1, -1), WpT)


@jax.jit
def kernel(b_x, indices, table, Wk, bk, Wl, bl, Wp, bp):
    tableT = table.T
    embsT = _make_sc_gather(N_NODES, EMBED_DIM, BATCH)(tableT, indices)
    WpT = Wp.T
    pfT = _mlp_a(b_x, Wk.T, bk, WpT, bp)
    outT = _mlp_b(pfT, embsT, Wl, bl, WpT)
    return outT.T


# speedup vs baseline: 1.0307x; 1.0307x over previous
import functools

import jax
import jax.numpy as jnp
from jax import lax
from jax.experimental import pallas as pl
from jax.experimental.pallas import tpu as pltpu
from jax.experimental.pallas import tpu_sc as plsc

N_NODES = 1000000
EMBED_DIM = 64
FEAT_DIM = 256
LABELS_SIZE = 64
BATCH = 4096

_LANES = 16
_TILE_W = 128
_CHUNK = 2
_RING = 7


@functools.cache
def _make_sc_gather(V, D, B):
    info = plsc.get_sparse_core_info()
    NC, NS = info.num_cores, info.num_subcores
    NW = NC * NS
    assert B % (8 * NW) == 0
    b_per_w = B // NW
    n_chunks = b_per_w // _CHUNK
    mesh = plsc.VectorSubcoreMesh(core_axis_name="c", subcore_axis_name="s")

    @functools.partial(
        pl.kernel,
        mesh=mesh,
        out_type=jax.ShapeDtypeStruct((D, B), jnp.float32),
        compiler_params=pltpu.CompilerParams(needs_layout_passes=False),
        scratch_types=[
            pltpu.VMEM((b_per_w,), jnp.int32),
            pltpu.VMEM((_RING, _CHUNK, D, _TILE_W), jnp.float32),
            pltpu.VMEM((D, b_per_w), jnp.float32),
        ] + [pltpu.SemaphoreType.DMA] * _RING,
    )
    def gather(tableT_hbm, idx_hbm, outT_hbm, idx_v, bufs, colsT_v, *sems):
        wid = lax.axis_index("s") * NC + lax.axis_index("c")
        base = pl.multiple_of(wid * b_per_w, b_per_w)
        pltpu.sync_copy(idx_hbm.at[pl.ds(base, b_per_w)], idx_v)

        tc_vecs = []
        c_vecs = []
        for q in range(b_per_w // _LANES):
            vec = idx_v[pl.ds(q * _LANES, _LANES)]
            tc_vecs.append(lax.shift_right_logical(vec, 7))
            c_vecs.append(lax.bitwise_and(vec, 127))

        iota = lax.broadcasted_iota(jnp.int32, (_LANES,), 0)

        def issue(g):
            slot = g % _RING
            cps = []
            for b in range(_CHUNK):
                j = g * _CHUNK + b
                q, k = j // _LANES, j % _LANES
                off = pl.multiple_of(tc_vecs[q][k] * _TILE_W, _TILE_W)
                cps.append(pltpu.async_copy(
                    tableT_hbm.at[:, pl.ds(off, _TILE_W)],
                    bufs.at[slot, b], sems[slot]))
            return cps

        pending = {g: issue(g) for g in range(min(_RING - 1, n_chunks))}
        for g in range(n_chunks):
            ahead = g + _RING - 1
            if ahead < n_chunks:
                pending[ahead] = issue(ahead)
            for c in pending.pop(g):
                c.wait()
            slot = g % _RING
            for b in range(_CHUNK):
                j = g * _CHUNK + b
                q, k = j // _LANES, j % _LANES
                cols = jnp.full((_LANES,), c_vecs[q][k], jnp.int32)
                outcols = jnp.full((_LANES,), j, jnp.int32)
                for s in range(D // _LANES):
                    rows = iota + (s * _LANES)
                    vals = plsc.load_gather(bufs.at[slot, b], [rows, cols])
                    plsc.store_scatter(colsT_v, [rows, outcols], vals)

        pltpu.sync_copy(colsT_v, outT_hbm.at[:, pl.ds(base, b_per_w)])

    return gather


def _mlp_a_body(bx_ref, wkT_ref, bk_ref, wpT_ref, bp_ref, pfT_ref):
    wpfT = wpT_ref[:, :LABELS_SIZE]
    h_f = jnp.maximum(
        lax.dot_general(bx_ref[...], wkT_ref[...],
                        dimension_numbers=(((1,), (1,)), ((), ())),
                        preferred_element_type=jnp.float32) + bk_ref[...],
        0.0)
    bp_col = jnp.reshape(bp_ref[...], (LABELS_SIZE, 1))
    pfT_ref[...] = lax.dot_general(
        wpfT, h_f,
        dimension_numbers=(((1,), (1,)), ((), ())),
        preferred_element_type=jnp.float32) + bp_col


def _mlp_b_body(pfT_ref, embsT_ref, wl_ref, bl_ref, wpT_ref, outT_ref):
    wpeT = wpT_ref[:, LABELS_SIZE:]
    h_e = jnp.maximum(
        lax.dot_general(embsT_ref[...], wl_ref[...],
                        dimension_numbers=(((0,), (0,)), ((), ())),
                        preferred_element_type=jnp.float32) + bl_ref[...],
        0.0)
    logitsT = pfT_ref[...] + lax.dot_general(
        wpeT, h_e,
        dimension_numbers=(((1,), (1,)), ((), ())),
        preferred_element_type=jnp.float32)
    m = jnp.max(logitsT, axis=0, keepdims=True)
    e = jnp.exp(logitsT - m)
    outT_ref[...] = e / jnp.sum(e, axis=0, keepdims=True)


def _mlp_a(b_x, WkT, bk, WpT, bp):
    B, F = b_x.shape
    BLK = 2048
    return pl.pallas_call(
        _mlp_a_body,
        grid=(B // BLK,),
        in_specs=[
            pl.BlockSpec((BLK, F), lambda i: (i, 0)),
            pl.BlockSpec((LABELS_SIZE, F), lambda i: (0, 0)),
            pl.BlockSpec((1, LABELS_SIZE), lambda i: (0, 0)),
            pl.BlockSpec((LABELS_SIZE, 2 * LABELS_SIZE), lambda i: (0, 0)),
            pl.BlockSpec((1, LABELS_SIZE), lambda i: (0, 0)),
        ],
        out_specs=pl.BlockSpec((LABELS_SIZE, BLK), lambda i: (0, i)),
        out_shape=jax.ShapeDtypeStruct((LABELS_SIZE, B), jnp.float32),
    )(b_x, WkT, bk.reshape(1, -1), WpT, bp.reshape(1, -1))


def _mlp_b(pfT, embsT, Wl, bl, WpT):
    _, B = pfT.shape
    BLK = 2048
    return pl.pallas_call(
        _mlp_b_body,
        grid=(B // BLK,),
        in_specs=[
            pl.BlockSpec((LABELS_SIZE, BLK), lambda i: (0, i)),
            pl.BlockSpec((EMBED_DIM, BLK), lambda i: (0, i)),
            pl.BlockSpec((EMBED_DIM, LABELS_SIZE), lambda i: (0, 0)),
            pl.BlockSpec((1, LABELS_SIZE), lambda i: (0, 0)),
            pl.BlockSpec((LABELS_SIZE, 2 * LABELS_SIZE), lambda i: (0, 0)),
        ],
        out_specs=pl.BlockSpec((LABELS_SIZE, BLK), lambda i: (0, i)),
        out_shape=jax.ShapeDtypeStruct((LABELS_SIZE, B), jnp.float32),
    )(pfT, embsT, Wl, bl.reshape(1, -1), WpT)


@jax.jit
def kernel(b_x, indices, table, Wk, bk, Wl, bl, Wp, bp):
    tableT = table.T
    embsT = _make_sc_gather(N_NODES, EMBED_DIM, BATCH)(tableT, indices)
    WpT = Wp.T
    pfT = _mlp_a(b_x, Wk.T, bk, WpT, bp)
    outT = _mlp_b(pfT, embsT, Wl, bl, WpT)
    return outT.T
